# baseline (speedup 1.0000x reference)
import functools

import jax
import jax.numpy as jnp
from jax import lax
from jax.experimental import pallas as pl
from jax.experimental.pallas import tpu as pltpu
from jax.experimental.pallas import tpu_sc as plsc


def kernel(x, W, H):
    B = x.shape[0]
    K = W.shape[1]
    uidx = x[:, 0].astype(jnp.int32)
    iidx = x[:, 1].astype(jnp.int32)

    info = plsc.get_sparse_core_info()
    NC, NS, L = info.num_cores, info.num_subcores, info.num_lanes
    NW = NC * NS
    bpw = B // NW
    nblk = bpw // L

    mesh = plsc.VectorSubcoreMesh(core_axis_name="c", subcore_axis_name="s")

    @functools.partial(
        pl.kernel,
        mesh=mesh,
        out_type=jax.ShapeDtypeStruct((B,), jnp.float32),
        scratch_types=[
            pltpu.VMEM((bpw,), jnp.int32),
            pltpu.VMEM((bpw,), jnp.int32),
            pltpu.VMEM((bpw, K), jnp.float32),
            pltpu.VMEM((bpw, K), jnp.float32),
            pltpu.VMEM((bpw,), jnp.float32),
            pltpu.SemaphoreType.DMA,
        ],
        compiler_params=pltpu.CompilerParams(
            needs_layout_passes=False, use_tc_tiling_on_sc=False
        ),
    )
    def mf_dot(w_hbm, h_hbm, u_hbm, i_hbm, out_hbm, u_v, i_v, w_v, h_v, o_v, sem):
        wid = lax.axis_index("s") * NC + lax.axis_index("c")
        base = wid * bpw
        pltpu.sync_copy(u_hbm.at[pl.ds(base, bpw)], u_v)
        pltpu.sync_copy(i_hbm.at[pl.ds(base, bpw)], i_v)
        cw = pltpu.async_copy(w_hbm.at[u_v], w_v, sem)
        ch = pltpu.async_copy(h_hbm.at[i_v], h_v, sem)
        cw.wait()
        ch.wait()

        lane = lax.iota(jnp.int32, L)

        def block(b, carry):
            rows = b * L + lane
            acc = jnp.zeros((L,), jnp.float32)
            for k in range(K):
                col = jnp.full((L,), k, jnp.int32)
                wcol = plsc.load_gather(w_v, [rows, col])
                hcol = plsc.load_gather(h_v, [rows, col])
                acc = acc + wcol * hcol
            o_v[pl.ds(b * L, L)] = 1.0 / (1.0 + jnp.exp(-acc))
            return carry

        lax.fori_loop(0, nblk, block, 0)
        pltpu.sync_copy(o_v, out_hbm.at[pl.ds(base, bpw)])

    return mf_dot(W, H, uidx, iidx)
